# baseline (speedup 1.0000x reference)
import functools

import jax
import jax.numpy as jnp
from jax import lax
from jax.experimental import pallas as pl
from jax.experimental.pallas import tpu as pltpu
from jax.experimental.pallas import tpu_sc as plsc

N = 10000
E = 320000
D = 128
HALF = D // 2
NTILES = 16
CHUNK = 80
EPT = E // NTILES
NCHUNK = EPT // CHUNK
ROWS_PER_TILE = N // NTILES


def _sc_segment_sum_plus_x(feat_halves, src2, dst2):
    mesh = plsc.VectorSubcoreMesh(core_axis_name="c", subcore_axis_name="s")

    @functools.partial(
        pl.kernel,
        mesh=mesh,
        compiler_params=pltpu.CompilerParams(use_tc_tiling_on_sc=False),
        out_type=jax.ShapeDtypeStruct((N, D), jnp.float32),
        scratch_types=[
            pltpu.VMEM_SHARED((N, HALF), jnp.float32),
            pltpu.VMEM((NCHUNK, CHUNK), jnp.int32),
            pltpu.VMEM((NCHUNK, CHUNK), jnp.int32),
            pltpu.VMEM((CHUNK, HALF), jnp.float32),
            pltpu.VMEM((CHUNK, HALF), jnp.float32),
            pltpu.SemaphoreType.DMA,
            pltpu.SemaphoreType.DMA,
        ],
    )
    def k(feat_hbm, src_hbm, dst_hbm, out_hbm,
          acc_sh, src_v, dst_v, rows0, rows1, sem0, sem1):
        cid = lax.axis_index("c")
        sid = lax.axis_index("s")
        r0 = sid * ROWS_PER_TILE
        c0 = cid * HALF
        pltpu.sync_copy(feat_hbm.at[cid, pl.ds(r0, ROWS_PER_TILE)],
                        acc_sh.at[pl.ds(r0, ROWS_PER_TILE)])
        pltpu.sync_copy(src_hbm.at[pl.ds(sid * NCHUNK, NCHUNK)], src_v)
        pltpu.sync_copy(dst_hbm.at[pl.ds(sid * NCHUNK, NCHUNK)], dst_v)
        plsc.subcore_barrier()

        table = feat_hbm.at[cid]
        pltpu.async_copy(table.at[src_v.at[0]], rows0, sem0)

        def body(k2, carry):
            j0 = 2 * k2
            j1 = j0 + 1
            pltpu.make_async_copy(table.at[src_v.at[j0]], rows0, sem0).wait()
            pltpu.async_copy(table.at[src_v.at[j1]], rows1, sem1)
            pltpu.sync_copy(rows0, acc_sh.at[dst_v.at[j0]], add=True)
            pltpu.make_async_copy(table.at[src_v.at[j1]], rows1, sem1).wait()
            jn = jnp.minimum(j0 + 2, NCHUNK - 1)
            pltpu.async_copy(table.at[src_v.at[jn]], rows0, sem0)
            pltpu.sync_copy(rows1, acc_sh.at[dst_v.at[j1]], add=True)
            return carry

        lax.fori_loop(0, NCHUNK // 2, body, 0)
        pltpu.make_async_copy(table.at[src_v.at[NCHUNK - 1]], rows0,
                              sem0).wait()
        plsc.subcore_barrier()
        pltpu.sync_copy(acc_sh.at[pl.ds(r0, ROWS_PER_TILE)],
                        out_hbm.at[pl.ds(r0, ROWS_PER_TILE), pl.ds(c0, HALF)])

    return k(feat_halves, src2, dst2)


def _tc_mlp_bn(pooled_plus_x, feature, eps, W1, b1, W2, b2, gamma, beta):
    def body(eps_ref, pp_ref, x_ref, w1_ref, b1_ref, w2_ref, b2_ref,
             g_ref, bt_ref, o_ref):
        y = pp_ref[...] + eps_ref[0] * x_ref[...]
        h = jnp.dot(y, w1_ref[...], preferred_element_type=jnp.float32)
        h = jnp.maximum(h + b1_ref[...], 0.0)
        h = jnp.dot(h, w2_ref[...], preferred_element_type=jnp.float32)
        h = h + b2_ref[...]
        mean = jnp.mean(h, axis=0, keepdims=True)
        d = h - mean
        var = jnp.mean(d * d, axis=0, keepdims=True)
        h = d * lax.rsqrt(var + 1e-5) * g_ref[...] + bt_ref[...]
        o_ref[...] = jnp.maximum(h, 0.0)

    vspec = pl.BlockSpec(memory_space=pltpu.VMEM)
    return pl.pallas_call(
        body,
        out_shape=jax.ShapeDtypeStruct((N, D), jnp.float32),
        in_specs=[pl.BlockSpec(memory_space=pltpu.SMEM)] + [vspec] * 8,
        out_specs=vspec,
    )(eps, pooled_plus_x, feature, W1, b1.reshape(1, D), W2,
      b2.reshape(1, D), gamma.reshape(1, D), beta.reshape(1, D))


def kernel(feature, edge_index, eps, W1, b1, W2, b2, gamma, beta):
    src2 = edge_index[0].reshape(E // CHUNK, CHUNK)
    dst2 = edge_index[1].reshape(E // CHUNK, CHUNK)
    feat_halves = jnp.stack([feature[:, :HALF], feature[:, HALF:]])
    pooled_plus_x = _sc_segment_sum_plus_x(feat_halves, src2, dst2)
    return _tc_mlp_bn(pooled_plus_x, feature, eps, W1, b1, W2, b2,
                      gamma, beta)


# speedup vs baseline: 1.3612x; 1.3612x over previous
import functools

import jax
import jax.numpy as jnp
from jax import lax
from jax.experimental import pallas as pl
from jax.experimental.pallas import tpu as pltpu
from jax.experimental.pallas import tpu_sc as plsc

N = 10000
E = 320000
D = 128
HALF = D // 2
NTILES = 16
CHUNK = 80
EPT = E // NTILES
NCHUNK = EPT // CHUNK
ROWS_PER_TILE = N // NTILES


def _sc_segment_sum_plus_x(feat_halves, src2, dst2):
    mesh = plsc.VectorSubcoreMesh(core_axis_name="c", subcore_axis_name="s")

    @functools.partial(
        pl.kernel,
        mesh=mesh,
        compiler_params=pltpu.CompilerParams(use_tc_tiling_on_sc=False),
        out_type=jax.ShapeDtypeStruct((N, D), jnp.float32),
        scratch_types=[
            pltpu.VMEM_SHARED((N, HALF), jnp.float32),
            pltpu.VMEM_SHARED((N, HALF), jnp.float32),
            pltpu.VMEM((NCHUNK, CHUNK), jnp.int32),
            pltpu.VMEM((NCHUNK, CHUNK), jnp.int32),
            pltpu.VMEM((CHUNK, HALF), jnp.float32),
            pltpu.VMEM((CHUNK, HALF), jnp.float32),
            pltpu.SemaphoreType.DMA,
            pltpu.SemaphoreType.DMA,
        ],
    )
    def k(feat_hbm, src_hbm, dst_hbm, out_hbm,
          feat_sh, acc_sh, src_v, dst_v, rows0, rows1, sem0, sem1):
        cid = lax.axis_index("c")
        sid = lax.axis_index("s")
        r0 = sid * ROWS_PER_TILE
        c0 = cid * HALF
        pltpu.sync_copy(feat_hbm.at[cid, pl.ds(r0, ROWS_PER_TILE)],
                        feat_sh.at[pl.ds(r0, ROWS_PER_TILE)])
        pltpu.sync_copy(feat_hbm.at[cid, pl.ds(r0, ROWS_PER_TILE)],
                        acc_sh.at[pl.ds(r0, ROWS_PER_TILE)])
        pltpu.sync_copy(src_hbm.at[pl.ds(sid * NCHUNK, NCHUNK)], src_v)
        pltpu.sync_copy(dst_hbm.at[pl.ds(sid * NCHUNK, NCHUNK)], dst_v)
        plsc.subcore_barrier()

        table = feat_sh
        pltpu.async_copy(table.at[src_v.at[0]], rows0, sem0)

        def body(k2, carry):
            j0 = 2 * k2
            j1 = j0 + 1
            pltpu.make_async_copy(table.at[src_v.at[j0]], rows0, sem0).wait()
            pltpu.async_copy(table.at[src_v.at[j1]], rows1, sem1)
            pltpu.sync_copy(rows0, acc_sh.at[dst_v.at[j0]], add=True)
            pltpu.make_async_copy(table.at[src_v.at[j1]], rows1, sem1).wait()
            jn = jnp.minimum(j0 + 2, NCHUNK - 1)
            pltpu.async_copy(table.at[src_v.at[jn]], rows0, sem0)
            pltpu.sync_copy(rows1, acc_sh.at[dst_v.at[j1]], add=True)
            return carry

        lax.fori_loop(0, NCHUNK // 2, body, 0)
        pltpu.make_async_copy(table.at[src_v.at[NCHUNK - 1]], rows0,
                              sem0).wait()
        plsc.subcore_barrier()
        pltpu.sync_copy(acc_sh.at[pl.ds(r0, ROWS_PER_TILE)],
                        out_hbm.at[pl.ds(r0, ROWS_PER_TILE), pl.ds(c0, HALF)])

    return k(feat_halves, src2, dst2)


def _tc_mlp_bn(pooled_plus_x, feature, eps, W1, b1, W2, b2, gamma, beta):
    def body(eps_ref, pp_ref, x_ref, w1_ref, b1_ref, w2_ref, b2_ref,
             g_ref, bt_ref, o_ref):
        y = pp_ref[...] + eps_ref[0] * x_ref[...]
        h = jnp.dot(y, w1_ref[...], preferred_element_type=jnp.float32)
        h = jnp.maximum(h + b1_ref[...], 0.0)
        h = jnp.dot(h, w2_ref[...], preferred_element_type=jnp.float32)
        h = h + b2_ref[...]
        mean = jnp.mean(h, axis=0, keepdims=True)
        d = h - mean
        var = jnp.mean(d * d, axis=0, keepdims=True)
        h = d * lax.rsqrt(var + 1e-5) * g_ref[...] + bt_ref[...]
        o_ref[...] = jnp.maximum(h, 0.0)

    vspec = pl.BlockSpec(memory_space=pltpu.VMEM)
    return pl.pallas_call(
        body,
        out_shape=jax.ShapeDtypeStruct((N, D), jnp.float32),
        in_specs=[pl.BlockSpec(memory_space=pltpu.SMEM)] + [vspec] * 8,
        out_specs=vspec,
    )(eps, pooled_plus_x, feature, W1, b1.reshape(1, D), W2,
      b2.reshape(1, D), gamma.reshape(1, D), beta.reshape(1, D))


def kernel(feature, edge_index, eps, W1, b1, W2, b2, gamma, beta):
    src2 = edge_index[0].reshape(E // CHUNK, CHUNK)
    dst2 = edge_index[1].reshape(E // CHUNK, CHUNK)
    feat_halves = jnp.stack([feature[:, :HALF], feature[:, HALF:]])
    pooled_plus_x = _sc_segment_sum_plus_x(feat_halves, src2, dst2)
    return _tc_mlp_bn(pooled_plus_x, feature, eps, W1, b1, W2, b2,
                      gamma, beta)


# speedup vs baseline: 1.7233x; 1.2661x over previous
import functools

import jax
import jax.numpy as jnp
from jax import lax
from jax.experimental import pallas as pl
from jax.experimental.pallas import tpu as pltpu
from jax.experimental.pallas import tpu_sc as plsc

N = 10000
E = 320000
D = 128
HALF = D // 2
NTILES = 16
CHUNK = 80
EPT = E // NTILES
NCHUNK = EPT // CHUNK
ROWS_PER_TILE = N // NTILES
RING = 5
LOOKAHEAD = 3


def _sc_segment_sum_plus_x(feat_halves, src2, dst2):
    mesh = plsc.VectorSubcoreMesh(core_axis_name="c", subcore_axis_name="s")

    @functools.partial(
        pl.kernel,
        mesh=mesh,
        compiler_params=pltpu.CompilerParams(use_tc_tiling_on_sc=False),
        out_type=jax.ShapeDtypeStruct((N, D), jnp.float32),
        scratch_types=[
            pltpu.VMEM_SHARED((N, HALF), jnp.float32),
            pltpu.VMEM((NCHUNK, CHUNK), jnp.int32),
            pltpu.VMEM((NCHUNK, CHUNK), jnp.int32),
            [pltpu.VMEM((CHUNK, HALF), jnp.float32)] * RING,
            [pltpu.SemaphoreType.DMA] * RING,
            [pltpu.SemaphoreType.DMA] * RING,
        ],
    )
    def k(feat_hbm, src_hbm, dst_hbm, out_hbm,
          acc_sh, src_v, dst_v, bufs, sg, ss):
        cid = lax.axis_index("c")
        sid = lax.axis_index("s")
        r0 = sid * ROWS_PER_TILE
        c0 = cid * HALF
        pltpu.sync_copy(feat_hbm.at[cid, pl.ds(r0, ROWS_PER_TILE)],
                        acc_sh.at[pl.ds(r0, ROWS_PER_TILE)])
        pltpu.sync_copy(src_hbm.at[pl.ds(sid * NCHUNK, NCHUNK)], src_v)
        pltpu.sync_copy(dst_hbm.at[pl.ds(sid * NCHUNK, NCHUNK)], dst_v)
        plsc.subcore_barrier()

        table = feat_hbm.at[cid]
        for m in range(LOOKAHEAD):
            pltpu.async_copy(table.at[src_v.at[m]], bufs[m], sg[m])

        def body(k2, carry):
            for i in range(RING):
                j = RING * k2 + i
                pltpu.make_async_copy(table.at[src_v.at[j]], bufs[i],
                                      sg[i]).wait()
                pltpu.async_copy(bufs[i], acc_sh.at[dst_v.at[j]], ss[i],
                                 add=True)
                m = (i + LOOKAHEAD) % RING

                def drain_prev_scatter():
                    pltpu.make_async_copy(bufs[m], acc_sh.at[dst_v.at[0]],
                                          ss[m]).wait()

                if i >= 2:
                    drain_prev_scatter()
                else:
                    pl.when(k2 > 0)(drain_prev_scatter)
                jn = jnp.minimum(j + LOOKAHEAD, NCHUNK - 1)
                pltpu.async_copy(table.at[src_v.at[jn]], bufs[m], sg[m])
            return carry

        lax.fori_loop(0, NCHUNK // RING, body, 0)
        for m in range(LOOKAHEAD):
            pltpu.make_async_copy(table.at[src_v.at[NCHUNK - 1]], bufs[m],
                                  sg[m]).wait()
        for m in (RING - 2, RING - 1):
            pltpu.make_async_copy(bufs[m], acc_sh.at[dst_v.at[0]],
                                  ss[m]).wait()
        plsc.subcore_barrier()
        pltpu.sync_copy(acc_sh.at[pl.ds(r0, ROWS_PER_TILE)],
                        out_hbm.at[pl.ds(r0, ROWS_PER_TILE), pl.ds(c0, HALF)])

    return k(feat_halves, src2, dst2)


def _tc_mlp_bn(pooled_plus_x, feature, eps, W1, b1, W2, b2, gamma, beta):
    def body(eps_ref, pp_ref, x_ref, w1_ref, b1_ref, w2_ref, b2_ref,
             g_ref, bt_ref, o_ref):
        y = pp_ref[...] + eps_ref[0] * x_ref[...]
        h = jnp.dot(y, w1_ref[...], preferred_element_type=jnp.float32)
        h = jnp.maximum(h + b1_ref[...], 0.0)
        h = jnp.dot(h, w2_ref[...], preferred_element_type=jnp.float32)
        h = h + b2_ref[...]
        mean = jnp.mean(h, axis=0, keepdims=True)
        d = h - mean
        var = jnp.mean(d * d, axis=0, keepdims=True)
        h = d * lax.rsqrt(var + 1e-5) * g_ref[...] + bt_ref[...]
        o_ref[...] = jnp.maximum(h, 0.0)

    vspec = pl.BlockSpec(memory_space=pltpu.VMEM)
    return pl.pallas_call(
        body,
        out_shape=jax.ShapeDtypeStruct((N, D), jnp.float32),
        in_specs=[pl.BlockSpec(memory_space=pltpu.SMEM)] + [vspec] * 8,
        out_specs=vspec,
    )(eps, pooled_plus_x, feature, W1, b1.reshape(1, D), W2,
      b2.reshape(1, D), gamma.reshape(1, D), beta.reshape(1, D))


def kernel(feature, edge_index, eps, W1, b1, W2, b2, gamma, beta):
    src2 = edge_index[0].reshape(E // CHUNK, CHUNK)
    dst2 = edge_index[1].reshape(E // CHUNK, CHUNK)
    feat_halves = jnp.stack([feature[:, :HALF], feature[:, HALF:]])
    pooled_plus_x = _sc_segment_sum_plus_x(feat_halves, src2, dst2)
    return _tc_mlp_bn(pooled_plus_x, feature, eps, W1, b1, W2, b2,
                      gamma, beta)
